# baseline (speedup 1.0000x reference)
import functools

import jax
import jax.numpy as jnp
from jax import lax
from jax.experimental import pallas as pl
from jax.experimental.pallas import tpu as pltpu
from jax.experimental.pallas import tpu_sc as plsc

_NC = 2
_NS = 16
_NW = _NC * _NS
_L = 16
_CH = 40


def _pack_pair(a, b):
    au = lax.bitcast_convert_type(a.astype(jnp.bfloat16), jnp.uint16)
    bu = lax.bitcast_convert_type(b.astype(jnp.bfloat16), jnp.uint16)
    w = au.astype(jnp.uint32) | (bu.astype(jnp.uint32) << 16)
    return lax.bitcast_convert_type(w, jnp.int32)


def _pre_body(x_ref, wk_ref, wqv_ref, bk_ref, bqv_ref, k_ref, qv_ref):
    xb = x_ref[...]
    h = wk_ref.shape[1]
    k_ref[...] = (
        jnp.dot(xb, wk_ref[...], preferred_element_type=jnp.float32)
        + bk_ref[...])
    qv = (jnp.dot(xb, wqv_ref[...], preferred_element_type=jnp.float32)
          + bqv_ref[...])
    qv_ref[...] = jnp.concatenate(
        [_pack_pair(qv[:, :h // 2], qv[:, h // 2:h]),
         _pack_pair(qv[:, h:h + h // 2], qv[:, h + h // 2:])], axis=1)


def _pre(x, w_key, w_qv, b_key, b_qv, blk):
    n, d = x.shape
    h = w_key.shape[1]
    grid = (n // blk,)
    return pl.pallas_call(
        _pre_body,
        grid=grid,
        in_specs=[
            pl.BlockSpec((blk, d), lambda i: (i, 0)),
            pl.BlockSpec((d, h), lambda i: (0, 0)),
            pl.BlockSpec((d, 2 * h), lambda i: (0, 0)),
            pl.BlockSpec((1, h), lambda i: (0, 0)),
            pl.BlockSpec((1, 2 * h), lambda i: (0, 0)),
        ],
        out_specs=[
            pl.BlockSpec((blk, h), lambda i: (i, 0)),
            pl.BlockSpec((blk, h), lambda i: (i, 0)),
        ],
        out_shape=[
            jax.ShapeDtypeStruct((n, h), jnp.float32),
            jax.ShapeDtypeStruct((n, h), jnp.int32),
        ],
    )(x, w_key, w_qv, b_key[None, :], b_qv[None, :])


def _post_body(x_ref, a0_ref, a1_ref, ws_ref, wl_ref, bs_ref, bl_ref, o_ref):
    a = (
        a0_ref[0]
        + a1_ref[0]
        + jnp.dot(x_ref[...], ws_ref[...], preferred_element_type=jnp.float32)
        + bs_ref[...]
    )
    a = jnp.where(a > 0, a, 0.01 * a)
    o_ref[...] = (
        jnp.dot(a, wl_ref[...], preferred_element_type=jnp.float32) + bl_ref[...]
    )


def _post(x, agg, w_skip, w_lin, b_s, b_l, blk):
    n, d = x.shape
    h = w_skip.shape[1]
    grid = (n // blk,)
    return pl.pallas_call(
        _post_body,
        grid=grid,
        in_specs=[
            pl.BlockSpec((blk, d), lambda i: (i, 0)),
            pl.BlockSpec((1, blk, h), lambda i: (0, i, 0)),
            pl.BlockSpec((1, blk, h), lambda i: (1, i, 0)),
            pl.BlockSpec((d, h), lambda i: (0, 0)),
            pl.BlockSpec((h, h), lambda i: (0, 0)),
            pl.BlockSpec((1, h), lambda i: (0, 0)),
            pl.BlockSpec((1, h), lambda i: (0, 0)),
        ],
        out_specs=pl.BlockSpec((blk, h), lambda i: (i, 0)),
        out_shape=jax.ShapeDtypeStruct((n, h), jnp.float32),
    )(x, agg, agg, w_skip, w_lin, b_s[None, :], b_l[None, :])


_NCHK = 6
_GD = 3


def _sc_edge_body(n, e, h, src_hbm, dst_hbm, k_hbm, qv_hbm, agg_hbm,
                  sidx, didx, dscat, kbuf, qvbuf, msgbuf, agg_sh,
                  g0, g1, g2, s0, s1, i0, i1):
    c = lax.axis_index("c")
    s = lax.axis_index("s")
    wid = c * _NS + s
    epw = e // _NW
    npair = epw // (2 * _NCHK * _CH)
    n_pad = agg_hbm.shape[1]
    rows_pt = n_pad // _NS
    r0 = s * rows_pt
    h8 = h // _L
    gsem = (g0, g1, g2)
    ssem = (s0, s1)
    isem = (i0, i1)

    def _zrow(i, _):
        for w in range(h8):
            msgbuf[0, i, pl.ds(w * _L, _L)] = jnp.zeros((_L,), jnp.float32)
        return 0
    lax.fori_loop(0, _CH, _zrow, 0)
    for m in range(rows_pt // _CH):
        pltpu.sync_copy(msgbuf.at[0], agg_sh.at[pl.ds(r0 + m * _CH, _CH)])
    plsc.subcore_barrier()

    def _issue_gather(isl, row, slot):
        pltpu.async_copy(k_hbm.at[didx.at[isl, row]], kbuf.at[slot],
                         gsem[slot])
        pltpu.async_copy(qv_hbm.at[sidx.at[isl, row]], qvbuf.at[slot],
                         gsem[slot])

    def _wait_gather(slot):
        pltpu.make_async_copy(k_hbm.at[didx.at[0, 0]], kbuf.at[slot],
                              gsem[slot]).wait()
        pltpu.make_async_copy(qv_hbm.at[sidx.at[0, 0]], qvbuf.at[slot],
                              gsem[slot]).wait()

    def _issue_idx(isl, sup):
        pltpu.async_copy(src_hbm.at[wid, sup], sidx.at[isl], isem[isl])
        pltpu.async_copy(dst_hbm.at[wid, sup], didx.at[isl], isem[isl])

    def _wait_idx(isl):
        pltpu.make_async_copy(src_hbm.at[wid, 0], sidx.at[isl],
                              isem[isl]).wait()
        pltpu.make_async_copy(dst_hbm.at[wid, 0], didx.at[isl],
                              isem[isl]).wait()

    def _compute(gslot, mslot):
        hh = h // 2

        def _unpack(w):
            lo = lax.bitcast_convert_type(w << 16, jnp.float32)
            hi = lax.bitcast_convert_type(w & jnp.int32(-65536), jnp.float32)
            return lo, hi

        def _edge(t, _):
            for w in range(hh // _L):
                klo = kbuf[gslot, t, pl.ds(w * _L, _L)]
                khi = kbuf[gslot, t, pl.ds(hh + w * _L, _L)]
                qlo, qhi = _unpack(qvbuf[gslot, t, pl.ds(w * _L, _L)])
                vlo, vhi = _unpack(qvbuf[gslot, t, pl.ds(hh + w * _L, _L)])
                msgbuf[mslot, t, pl.ds(w * _L, _L)] = (
                    jnp.maximum(klo + qlo, 0.0) * vlo)
                msgbuf[mslot, t, pl.ds(hh + w * _L, _L)] = (
                    jnp.maximum(khi + qhi, 0.0) * vhi)
            return 0
        lax.fori_loop(0, _CH, _edge, 0)

    def _issue_scatter(m):
        pltpu.async_copy(msgbuf.at[m], agg_sh.at[dscat.at[m]], ssem[m],
                         add=True)

    def _wait_scatter(m):
        pltpu.make_async_copy(msgbuf.at[m], agg_sh.at[dscat.at[m]],
                              ssem[m]).wait()

    _offs = sorted({o for w in range(_CH // _L) for o in (w * _L,)}
                   | {_CH - _L})

    def _copy_scatter_idx(isl, row, m):
        for o in _offs:
            dscat[m, pl.ds(o, _L)] = didx[isl, row, pl.ds(o, _L)]

    pltpu.sync_copy(src_hbm.at[wid, 0], sidx.at[0])
    pltpu.sync_copy(dst_hbm.at[wid, 0], didx.at[0])
    _issue_idx(1, 1)
    _issue_gather(0, 0, 0)
    _issue_gather(0, 1, 1)

    def _pair(p, _):
        for u in range(2 * _NCHK):
            isl, row, g, m = u // _NCHK, u % _NCHK, u % _GD, u % 2
            if u == _NCHK - 2:
                _wait_idx(1)
            if u + 2 < 2 * _NCHK:
                j = u + 2
                _issue_gather(j // _NCHK, j % _NCHK, j % _GD)
            else:
                @pl.when(p < npair - 1)
                def _():
                    if u == 2 * _NCHK - 2:
                        _wait_idx(0)
                    _issue_gather(0, u - (2 * _NCHK - 2), (u + 2) % _GD)
            _wait_gather(g)
            if u >= 2:
                _wait_scatter(m)
            else:
                @pl.when(p > 0)
                def _():
                    _wait_scatter(m)
            _compute(g, m)
            _copy_scatter_idx(isl, row, m)
            _issue_scatter(m)
            if u == _NCHK - 1:
                @pl.when(p < npair - 1)
                def _():
                    _issue_idx(0, 2 * p + 2)
        @pl.when(p < npair - 1)
        def _():
            _issue_idx(1, 2 * p + 3)
        return 0
    lax.fori_loop(0, npair, _pair, 0)
    _wait_scatter(0)
    _wait_scatter(1)
    plsc.subcore_barrier()

    for m in range(rows_pt // _CH):
        pltpu.sync_copy(agg_sh.at[pl.ds(r0 + m * _CH, _CH)], msgbuf.at[0])
        pltpu.sync_copy(msgbuf.at[0], agg_hbm.at[c, pl.ds(r0 + m * _CH, _CH)])


def _sc_edge(src, dst, k_table, qv_table):
    n, h = k_table.shape
    e = src.shape[0]
    n_pad = -(-n // (_NS * _CH)) * (_NS * _CH)
    epw = e // _NW
    nsup = epw // (_NCHK * _CH)
    src4 = src.reshape(_NW, nsup, _NCHK, _CH)
    dst4 = dst.reshape(_NW, nsup, _NCHK, _CH)
    body = functools.partial(_sc_edge_body, n, e, h)
    mesh = plsc.VectorSubcoreMesh(core_axis_name="c", subcore_axis_name="s")
    f = pl.kernel(
        body,
        out_type=jax.ShapeDtypeStruct((_NC, n_pad, h), jnp.float32),
        mesh=mesh,
        scratch_types=[
            pltpu.VMEM((2, _NCHK, _CH), jnp.int32),
            pltpu.VMEM((2, _NCHK, _CH), jnp.int32),
            pltpu.VMEM((2, _CH), jnp.int32),
            pltpu.VMEM((_GD, _CH, h), jnp.float32),
            pltpu.VMEM((_GD, _CH, h), jnp.int32),
            pltpu.VMEM((2, _CH, h), jnp.float32),
            pltpu.VMEM_SHARED((n_pad, h), jnp.float32),
            pltpu.SemaphoreType.DMA,
            pltpu.SemaphoreType.DMA,
            pltpu.SemaphoreType.DMA,
            pltpu.SemaphoreType.DMA,
            pltpu.SemaphoreType.DMA,
            pltpu.SemaphoreType.DMA,
            pltpu.SemaphoreType.DMA,
        ],
    )
    return f(src4, dst4, k_table, qv_table)


def kernel(x, edge_index, W_key, b_key, W_query, b_query, W_value, b_value,
           W_skip, b_skip, bias, W_lin, b_lin):
    n, d = x.shape
    w_qv = jnp.concatenate([W_query, W_value], axis=1)
    b_qv = jnp.concatenate([b_query, b_value], axis=0)
    n_pad = -(-n // (_NS * _CH)) * (_NS * _CH)
    x_pad = jnp.concatenate(
        [x, jnp.zeros((n_pad - n, d), jnp.float32)], axis=0)
    e = edge_index.shape[1]
    grain = _NW * 2 * _NCHK * _CH
    e_pad = -(-e // grain) * grain
    src = jnp.concatenate(
        [edge_index[0], jnp.zeros((e_pad - e,), jnp.int32)])
    dst = jnp.concatenate(
        [edge_index[1],
         n + jnp.arange(e_pad - e, dtype=jnp.int32) % (n_pad - n)])
    k_table, qv_table = _pre(x_pad, W_key, w_qv, b_key, b_qv, blk=512)
    agg = _sc_edge(src, dst, k_table, qv_table)
    return _post(x, agg, W_skip, W_lin, b_skip + bias, b_lin, blk=400)


# speedup vs baseline: 1.3419x; 1.3419x over previous
import functools

import jax
import jax.numpy as jnp
from jax import lax
from jax.experimental import pallas as pl
from jax.experimental.pallas import tpu as pltpu
from jax.experimental.pallas import tpu_sc as plsc

_NC = 2
_NS = 16
_NW = _NC * _NS
_L = 16
_CH = 40


def _pack_pair(a, b):
    au = lax.bitcast_convert_type(a.astype(jnp.bfloat16), jnp.uint16)
    bu = lax.bitcast_convert_type(b.astype(jnp.bfloat16), jnp.uint16)
    w = au.astype(jnp.uint32) | (bu.astype(jnp.uint32) << 16)
    return lax.bitcast_convert_type(w, jnp.int32)


def _pre_body(x_ref, wk_ref, wqv_ref, bk_ref, bqv_ref, k_ref, qv_ref):
    xb = x_ref[...]
    h = wk_ref.shape[1]
    k_ref[...] = (
        jnp.dot(xb, wk_ref[...], preferred_element_type=jnp.float32)
        + bk_ref[...])
    qv = (jnp.dot(xb, wqv_ref[...], preferred_element_type=jnp.float32)
          + bqv_ref[...])
    qv_ref[...] = jnp.concatenate(
        [_pack_pair(qv[:, :h // 2], qv[:, h // 2:h]),
         _pack_pair(qv[:, h:h + h // 2], qv[:, h + h // 2:])], axis=1)


def _pre(x, w_key, w_qv, b_key, b_qv, blk):
    n, d = x.shape
    h = w_key.shape[1]
    grid = (n // blk,)
    return pl.pallas_call(
        _pre_body,
        grid=grid,
        in_specs=[
            pl.BlockSpec((blk, d), lambda i: (i, 0)),
            pl.BlockSpec((d, h), lambda i: (0, 0)),
            pl.BlockSpec((d, 2 * h), lambda i: (0, 0)),
            pl.BlockSpec((1, h), lambda i: (0, 0)),
            pl.BlockSpec((1, 2 * h), lambda i: (0, 0)),
        ],
        out_specs=[
            pl.BlockSpec((blk, h), lambda i: (i, 0)),
            pl.BlockSpec((blk, h), lambda i: (i, 0)),
        ],
        out_shape=[
            jax.ShapeDtypeStruct((n, h), jnp.float32),
            jax.ShapeDtypeStruct((n, h), jnp.int32),
        ],
    )(x, w_key, w_qv, b_key[None, :], b_qv[None, :])


def _post_body(x_ref, a0_ref, a1_ref, ws_ref, wl_ref, bs_ref, bl_ref, o_ref):
    a = (
        a0_ref[0]
        + a1_ref[0]
        + jnp.dot(x_ref[...], ws_ref[...], preferred_element_type=jnp.float32)
        + bs_ref[...]
    )
    a = jnp.where(a > 0, a, 0.01 * a)
    o_ref[...] = (
        jnp.dot(a, wl_ref[...], preferred_element_type=jnp.float32) + bl_ref[...]
    )


def _post(x, agg, w_skip, w_lin, b_s, b_l, blk):
    n, d = x.shape
    h = w_skip.shape[1]
    grid = (n // blk,)
    return pl.pallas_call(
        _post_body,
        grid=grid,
        in_specs=[
            pl.BlockSpec((blk, d), lambda i: (i, 0)),
            pl.BlockSpec((1, blk, h), lambda i: (0, i, 0)),
            pl.BlockSpec((1, blk, h), lambda i: (1, i, 0)),
            pl.BlockSpec((d, h), lambda i: (0, 0)),
            pl.BlockSpec((h, h), lambda i: (0, 0)),
            pl.BlockSpec((1, h), lambda i: (0, 0)),
            pl.BlockSpec((1, h), lambda i: (0, 0)),
        ],
        out_specs=pl.BlockSpec((blk, h), lambda i: (i, 0)),
        out_shape=jax.ShapeDtypeStruct((n, h), jnp.float32),
    )(x, agg, agg, w_skip, w_lin, b_s[None, :], b_l[None, :])


_NCHK = 25
_GD = 3


def _sc_edge_body(n, e, h, src_hbm, dst_hbm, k_hbm, qv_hbm, agg_hbm,
                  sidx, didx, kbuf, qvbuf, msgbuf, agg_sh,
                  g0, g1, g2, s0, s1):
    c = lax.axis_index("c")
    s = lax.axis_index("s")
    wid = c * _NS + s
    epw = e // _NW
    nsc = epw // (_NCHK * _CH)
    n_pad = agg_hbm.shape[1]
    rows_pt = n_pad // _NS
    r0 = s * rows_pt
    h8 = h // _L
    gsem = (g0, g1, g2)
    ssem = (s0, s1)

    def _zrow(i, _):
        for w in range(h8):
            msgbuf[0, i, pl.ds(w * _L, _L)] = jnp.zeros((_L,), jnp.float32)
        return 0
    lax.fori_loop(0, _CH, _zrow, 0)
    for m in range(rows_pt // _CH):
        pltpu.sync_copy(msgbuf.at[0], agg_sh.at[pl.ds(r0 + m * _CH, _CH)])
    plsc.subcore_barrier()

    def _issue_gather(j, slot):
        pltpu.async_copy(k_hbm.at[didx.at[j]], kbuf.at[slot], gsem[slot])
        pltpu.async_copy(qv_hbm.at[sidx.at[j]], qvbuf.at[slot], gsem[slot])

    def _wait_gather(slot):
        pltpu.make_async_copy(k_hbm.at[didx.at[0]], kbuf.at[slot],
                              gsem[slot]).wait()
        pltpu.make_async_copy(qv_hbm.at[sidx.at[0]], qvbuf.at[slot],
                              gsem[slot]).wait()

    def _compute(gslot, mslot):
        hh = h // 2

        def _unpack(w):
            lo = lax.bitcast_convert_type(w << 16, jnp.float32)
            hi = lax.bitcast_convert_type(w & jnp.int32(-65536), jnp.float32)
            return lo, hi

        def _edge(t, _):
            for w in range(hh // _L):
                klo = kbuf[gslot, t, pl.ds(w * _L, _L)]
                khi = kbuf[gslot, t, pl.ds(hh + w * _L, _L)]
                qlo, qhi = _unpack(qvbuf[gslot, t, pl.ds(w * _L, _L)])
                vlo, vhi = _unpack(qvbuf[gslot, t, pl.ds(hh + w * _L, _L)])
                msgbuf[mslot, t, pl.ds(w * _L, _L)] = (
                    jnp.maximum(klo + qlo, 0.0) * vlo)
                msgbuf[mslot, t, pl.ds(hh + w * _L, _L)] = (
                    jnp.maximum(khi + qhi, 0.0) * vhi)
            return 0
        lax.fori_loop(0, _CH, _edge, 0)

    def _issue_scatter(j, slot):
        pltpu.async_copy(msgbuf.at[slot], agg_sh.at[didx.at[j]], ssem[slot],
                         add=True)

    def _wait_scatter(slot):
        pltpu.make_async_copy(msgbuf.at[slot], agg_sh.at[didx.at[0]],
                              ssem[slot]).wait()

    def _super(si, _):
        pltpu.sync_copy(src_hbm.at[wid, si], sidx)
        pltpu.sync_copy(dst_hbm.at[wid, si], didx)
        _issue_gather(0, 0)
        _issue_gather(1, 1)
        for c in range(_NCHK):
            if c + 2 < _NCHK:
                _issue_gather(c + 2, (c + 2) % _GD)
            _wait_gather(c % _GD)
            m = c % 2
            if c >= 2:
                _wait_scatter(m)
            _compute(c % _GD, m)
            _issue_scatter(c, m)
        _wait_scatter(0)
        _wait_scatter(1)
        return 0
    lax.fori_loop(0, nsc, _super, 0)
    plsc.subcore_barrier()

    for m in range(rows_pt // _CH):
        pltpu.sync_copy(agg_sh.at[pl.ds(r0 + m * _CH, _CH)], msgbuf.at[0])
        pltpu.sync_copy(msgbuf.at[0], agg_hbm.at[c, pl.ds(r0 + m * _CH, _CH)])


def _sc_edge(src, dst, k_table, qv_table):
    n, h = k_table.shape
    e = src.shape[0]
    n_pad = -(-n // (_NS * _CH)) * (_NS * _CH)
    epw = e // _NW
    nsc = epw // (_NCHK * _CH)
    src4 = src.reshape(_NW, nsc, _NCHK, _CH)
    dst4 = dst.reshape(_NW, nsc, _NCHK, _CH)
    body = functools.partial(_sc_edge_body, n, e, h)
    mesh = plsc.VectorSubcoreMesh(core_axis_name="c", subcore_axis_name="s")
    f = pl.kernel(
        body,
        out_type=jax.ShapeDtypeStruct((_NC, n_pad, h), jnp.float32),
        mesh=mesh,
        scratch_types=[
            pltpu.VMEM((_NCHK, _CH), jnp.int32),
            pltpu.VMEM((_NCHK, _CH), jnp.int32),
            pltpu.VMEM((_GD, _CH, h), jnp.float32),
            pltpu.VMEM((_GD, _CH, h), jnp.int32),
            pltpu.VMEM((2, _CH, h), jnp.float32),
            pltpu.VMEM_SHARED((n_pad, h), jnp.float32),
            pltpu.SemaphoreType.DMA,
            pltpu.SemaphoreType.DMA,
            pltpu.SemaphoreType.DMA,
            pltpu.SemaphoreType.DMA,
            pltpu.SemaphoreType.DMA,
        ],
    )
    return f(src4, dst4, k_table, qv_table)


def kernel(x, edge_index, W_key, b_key, W_query, b_query, W_value, b_value,
           W_skip, b_skip, bias, W_lin, b_lin):
    w_qv = jnp.concatenate([W_query, W_value], axis=1)
    b_qv = jnp.concatenate([b_query, b_value], axis=0)
    k_table, qv_table = _pre(x, W_key, w_qv, b_key, b_qv, blk=400)
    agg = _sc_edge(edge_index[0], edge_index[1], k_table, qv_table)
    return _post(x, agg, W_skip, W_lin, b_skip + bias, b_lin, blk=400)


# speedup vs baseline: 1.4604x; 1.0883x over previous
import functools

import jax
import jax.numpy as jnp
from jax import lax
from jax.experimental import pallas as pl
from jax.experimental.pallas import tpu as pltpu
from jax.experimental.pallas import tpu_sc as plsc

_NC = 2
_NS = 16
_NW = _NC * _NS
_L = 16
_CH = 40


def _pack_pair(a, b):
    au = lax.bitcast_convert_type(a.astype(jnp.bfloat16), jnp.uint16)
    bu = lax.bitcast_convert_type(b.astype(jnp.bfloat16), jnp.uint16)
    w = au.astype(jnp.uint32) | (bu.astype(jnp.uint32) << 16)
    return lax.bitcast_convert_type(w, jnp.int32)


def _pre_body(x_ref, wk_ref, wqv_ref, bk_ref, bqv_ref, k_ref, qv_ref):
    xb = x_ref[...]
    h = wk_ref.shape[1]
    k_ref[...] = (
        jnp.dot(xb, wk_ref[...], preferred_element_type=jnp.float32)
        + bk_ref[...])
    qv = (jnp.dot(xb, wqv_ref[...], preferred_element_type=jnp.float32)
          + bqv_ref[...])
    qv_ref[...] = jnp.concatenate(
        [_pack_pair(qv[:, :h // 2], qv[:, h // 2:h]),
         _pack_pair(qv[:, h:h + h // 2], qv[:, h + h // 2:])], axis=1)


def _pre(x, w_key, w_qv, b_key, b_qv, blk):
    n, d = x.shape
    h = w_key.shape[1]
    grid = (n // blk,)
    return pl.pallas_call(
        _pre_body,
        grid=grid,
        in_specs=[
            pl.BlockSpec((blk, d), lambda i: (i, 0)),
            pl.BlockSpec((d, h), lambda i: (0, 0)),
            pl.BlockSpec((d, 2 * h), lambda i: (0, 0)),
            pl.BlockSpec((1, h), lambda i: (0, 0)),
            pl.BlockSpec((1, 2 * h), lambda i: (0, 0)),
        ],
        out_specs=[
            pl.BlockSpec((blk, h), lambda i: (i, 0)),
            pl.BlockSpec((blk, h), lambda i: (i, 0)),
        ],
        out_shape=[
            jax.ShapeDtypeStruct((n, h), jnp.float32),
            jax.ShapeDtypeStruct((n, h), jnp.int32),
        ],
    )(x, w_key, w_qv, b_key[None, :], b_qv[None, :])


def _post_body(x_ref, a0_ref, a1_ref, ws_ref, wl_ref, bs_ref, bl_ref, o_ref):
    a = (
        a0_ref[0]
        + a1_ref[0]
        + jnp.dot(x_ref[...], ws_ref[...], preferred_element_type=jnp.float32)
        + bs_ref[...]
    )
    a = jnp.where(a > 0, a, 0.01 * a)
    o_ref[...] = (
        jnp.dot(a, wl_ref[...], preferred_element_type=jnp.float32) + bl_ref[...]
    )


def _post(x, agg, w_skip, w_lin, b_s, b_l, blk):
    n, d = x.shape
    h = w_skip.shape[1]
    grid = (n // blk,)
    return pl.pallas_call(
        _post_body,
        grid=grid,
        in_specs=[
            pl.BlockSpec((blk, d), lambda i: (i, 0)),
            pl.BlockSpec((1, blk, h), lambda i: (0, i, 0)),
            pl.BlockSpec((1, blk, h), lambda i: (1, i, 0)),
            pl.BlockSpec((d, h), lambda i: (0, 0)),
            pl.BlockSpec((h, h), lambda i: (0, 0)),
            pl.BlockSpec((1, h), lambda i: (0, 0)),
            pl.BlockSpec((1, h), lambda i: (0, 0)),
        ],
        out_specs=pl.BlockSpec((blk, h), lambda i: (i, 0)),
        out_shape=jax.ShapeDtypeStruct((n, h), jnp.float32),
    )(x, agg, agg, w_skip, w_lin, b_s[None, :], b_l[None, :])


_NCHK = 25
_GD = 3


def _sc_edge_body(n, e, h, src_hbm, dst_hbm, k_hbm, qv_hbm, agg_hbm,
                  sidx, didx, kbuf, qvbuf, msgbuf, agg_sh,
                  g0, g1, g2, s0, s1, i0):
    c = lax.axis_index("c")
    s = lax.axis_index("s")
    wid = c * _NS + s
    epw = e // _NW
    nsc = epw // (_NCHK * _CH)
    n_pad = agg_hbm.shape[1]
    rows_pt = n_pad // _NS
    r0 = s * rows_pt
    h8 = h // _L
    gsem = (g0, g1, g2)
    ssem = (s0, s1)

    def _zrow(i, _):
        for w in range(h8):
            msgbuf[0, i, pl.ds(w * _L, _L)] = jnp.zeros((_L,), jnp.float32)
        return 0
    lax.fori_loop(0, _CH, _zrow, 0)
    for m in range(rows_pt // _CH):
        pltpu.sync_copy(msgbuf.at[0], agg_sh.at[pl.ds(r0 + m * _CH, _CH)])
    plsc.subcore_barrier()

    def _issue_gather(j, slot):
        pltpu.async_copy(k_hbm.at[didx.at[j]], kbuf.at[slot], gsem[slot])
        pltpu.async_copy(qv_hbm.at[sidx.at[j]], qvbuf.at[slot], gsem[slot])

    def _wait_gather(slot):
        pltpu.make_async_copy(k_hbm.at[didx.at[0]], kbuf.at[slot],
                              gsem[slot]).wait()
        pltpu.make_async_copy(qv_hbm.at[sidx.at[0]], qvbuf.at[slot],
                              gsem[slot]).wait()

    def _compute(gslot, mslot):
        hh = h // 2

        def _unpack(w):
            lo = lax.bitcast_convert_type(w << 16, jnp.float32)
            hi = lax.bitcast_convert_type(w & jnp.int32(-65536), jnp.float32)
            return lo, hi

        def _edge(t, _):
            for w in range(hh // _L):
                klo = kbuf[gslot, t, pl.ds(w * _L, _L)]
                khi = kbuf[gslot, t, pl.ds(hh + w * _L, _L)]
                qlo, qhi = _unpack(qvbuf[gslot, t, pl.ds(w * _L, _L)])
                vlo, vhi = _unpack(qvbuf[gslot, t, pl.ds(hh + w * _L, _L)])
                msgbuf[mslot, t, pl.ds(w * _L, _L)] = (
                    jnp.maximum(klo + qlo, 0.0) * vlo)
                msgbuf[mslot, t, pl.ds(hh + w * _L, _L)] = (
                    jnp.maximum(khi + qhi, 0.0) * vhi)
            return 0
        lax.fori_loop(0, _CH, _edge, 0)

    def _issue_scatter(j, slot):
        pltpu.async_copy(msgbuf.at[slot], agg_sh.at[didx.at[j]], ssem[slot],
                         add=True)

    def _wait_scatter(slot):
        pltpu.make_async_copy(msgbuf.at[slot], agg_sh.at[didx.at[0]],
                              ssem[slot]).wait()

    def _super(si, _):
        ca = pltpu.async_copy(src_hbm.at[wid, si], sidx, i0)
        cb = pltpu.async_copy(dst_hbm.at[wid, si], didx, i0)
        ca.wait()
        cb.wait()
        _issue_gather(0, 0)
        _issue_gather(1, 1)
        for c in range(_NCHK):
            if c + 2 < _NCHK:
                _issue_gather(c + 2, (c + 2) % _GD)
            _wait_gather(c % _GD)
            m = c % 2
            if c >= 2:
                _wait_scatter(m)
            _compute(c % _GD, m)
            _issue_scatter(c, m)
        _wait_scatter(0)
        _wait_scatter(1)
        return 0
    lax.fori_loop(0, nsc, _super, 0)
    plsc.subcore_barrier()

    for m in range(rows_pt // _CH):
        pltpu.sync_copy(agg_sh.at[pl.ds(r0 + m * _CH, _CH)], msgbuf.at[0])
        pltpu.sync_copy(msgbuf.at[0], agg_hbm.at[c, pl.ds(r0 + m * _CH, _CH)])


def _sc_edge(src, dst, k_table, qv_table):
    n, h = k_table.shape
    e = src.shape[0]
    n_pad = -(-n // (_NS * _CH)) * (_NS * _CH)
    epw = e // _NW
    nsc = epw // (_NCHK * _CH)
    src4 = src.reshape(_NW, nsc, _NCHK, _CH)
    dst4 = dst.reshape(_NW, nsc, _NCHK, _CH)
    body = functools.partial(_sc_edge_body, n, e, h)
    mesh = plsc.VectorSubcoreMesh(core_axis_name="c", subcore_axis_name="s")
    f = pl.kernel(
        body,
        out_type=jax.ShapeDtypeStruct((_NC, n_pad, h), jnp.float32),
        mesh=mesh,
        scratch_types=[
            pltpu.VMEM((_NCHK, _CH), jnp.int32),
            pltpu.VMEM((_NCHK, _CH), jnp.int32),
            pltpu.VMEM((_GD, _CH, h), jnp.float32),
            pltpu.VMEM((_GD, _CH, h), jnp.int32),
            pltpu.VMEM((2, _CH, h), jnp.float32),
            pltpu.VMEM_SHARED((n_pad, h), jnp.float32),
            pltpu.SemaphoreType.DMA,
            pltpu.SemaphoreType.DMA,
            pltpu.SemaphoreType.DMA,
            pltpu.SemaphoreType.DMA,
            pltpu.SemaphoreType.DMA,
            pltpu.SemaphoreType.DMA,
        ],
    )
    return f(src4, dst4, k_table, qv_table)


def kernel(x, edge_index, W_key, b_key, W_query, b_query, W_value, b_value,
           W_skip, b_skip, bias, W_lin, b_lin):
    w_qv = jnp.concatenate([W_query, W_value], axis=1)
    b_qv = jnp.concatenate([b_query, b_value], axis=0)
    k_table, qv_table = _pre(x, W_key, w_qv, b_key, b_qv, blk=1000)
    agg = _sc_edge(edge_index[0], edge_index[1], k_table, qv_table)
    return _post(x, agg, W_skip, W_lin, b_skip + bias, b_lin, blk=1000)


# speedup vs baseline: 1.4966x; 1.0248x over previous
import functools

import jax
import jax.numpy as jnp
from jax import lax
from jax.experimental import pallas as pl
from jax.experimental.pallas import tpu as pltpu
from jax.experimental.pallas import tpu_sc as plsc

_NC = 2
_NS = 16
_NW = _NC * _NS
_L = 16
_CH = 40


def _pack_pair(a, b):
    au = lax.bitcast_convert_type(a.astype(jnp.bfloat16), jnp.uint16)
    bu = lax.bitcast_convert_type(b.astype(jnp.bfloat16), jnp.uint16)
    w = au.astype(jnp.uint32) | (bu.astype(jnp.uint32) << 16)
    return lax.bitcast_convert_type(w, jnp.int32)


def _pre_body(x_ref, wk_ref, wqv_ref, bk_ref, bqv_ref, k_ref, qv_ref):
    xb = x_ref[...]
    h = wk_ref.shape[1]
    k_ref[...] = (
        jnp.dot(xb, wk_ref[...], preferred_element_type=jnp.float32)
        + bk_ref[...])
    qv = (jnp.dot(xb, wqv_ref[...], preferred_element_type=jnp.float32)
          + bqv_ref[...])
    qv_ref[...] = jnp.concatenate(
        [_pack_pair(qv[:, :h // 2], qv[:, h // 2:h]),
         _pack_pair(qv[:, h:h + h // 2], qv[:, h + h // 2:])], axis=1)


def _pre(x, w_key, w_qv, b_key, b_qv, blk):
    n, d = x.shape
    h = w_key.shape[1]
    grid = (n // blk,)
    return pl.pallas_call(
        _pre_body,
        grid=grid,
        in_specs=[
            pl.BlockSpec((blk, d), lambda i: (i, 0)),
            pl.BlockSpec((d, h), lambda i: (0, 0)),
            pl.BlockSpec((d, 2 * h), lambda i: (0, 0)),
            pl.BlockSpec((1, h), lambda i: (0, 0)),
            pl.BlockSpec((1, 2 * h), lambda i: (0, 0)),
        ],
        out_specs=[
            pl.BlockSpec((blk, h), lambda i: (i, 0)),
            pl.BlockSpec((blk, h), lambda i: (i, 0)),
        ],
        out_shape=[
            jax.ShapeDtypeStruct((n, h), jnp.float32),
            jax.ShapeDtypeStruct((n, h), jnp.int32),
        ],
    )(x, w_key, w_qv, b_key[None, :], b_qv[None, :])


def _post_body(x_ref, a0_ref, a1_ref, ws_ref, wl_ref, bs_ref, bl_ref, o_ref):
    a = (
        a0_ref[0]
        + a1_ref[0]
        + jnp.dot(x_ref[...], ws_ref[...], preferred_element_type=jnp.float32)
        + bs_ref[...]
    )
    a = jnp.where(a > 0, a, 0.01 * a)
    o_ref[...] = (
        jnp.dot(a, wl_ref[...], preferred_element_type=jnp.float32) + bl_ref[...]
    )


def _post(x, agg, w_skip, w_lin, b_s, b_l, blk):
    n, d = x.shape
    h = w_skip.shape[1]
    grid = (n // blk,)
    return pl.pallas_call(
        _post_body,
        grid=grid,
        in_specs=[
            pl.BlockSpec((blk, d), lambda i: (i, 0)),
            pl.BlockSpec((1, blk, h), lambda i: (0, i, 0)),
            pl.BlockSpec((1, blk, h), lambda i: (1, i, 0)),
            pl.BlockSpec((d, h), lambda i: (0, 0)),
            pl.BlockSpec((h, h), lambda i: (0, 0)),
            pl.BlockSpec((1, h), lambda i: (0, 0)),
            pl.BlockSpec((1, h), lambda i: (0, 0)),
        ],
        out_specs=pl.BlockSpec((blk, h), lambda i: (i, 0)),
        out_shape=jax.ShapeDtypeStruct((n, h), jnp.float32),
    )(x, agg, agg, w_skip, w_lin, b_s[None, :], b_l[None, :])


_NCHK = 25
_GD = 3


def _sc_edge_body(n, e, h, src_hbm, dst_hbm, k_hbm, qv_hbm, agg_hbm,
                  sidx, didx, kbuf, qvbuf, msgbuf, agg_sh,
                  g0, g1, g2, s0, s1, i0):
    c = lax.axis_index("c")
    s = lax.axis_index("s")
    wid = c * _NS + s
    epw = e // _NW
    nsc = epw // (_NCHK * _CH)
    n_pad = agg_hbm.shape[1]
    rows_pt = n_pad // _NS
    r0 = s * rows_pt
    h8 = h // _L
    gsem = (g0, g1, g2)
    ssem = (s0, s1)

    def _zrow(i, _):
        for w in range(h8):
            msgbuf[0, i, pl.ds(w * _L, _L)] = jnp.zeros((_L,), jnp.float32)
        return 0
    lax.fori_loop(0, _CH, _zrow, 0)
    for m in range(rows_pt // _CH):
        pltpu.sync_copy(msgbuf.at[0], agg_sh.at[pl.ds(r0 + m * _CH, _CH)])
    plsc.subcore_barrier()

    def _issue_gather(j, slot):
        pltpu.async_copy(k_hbm.at[didx.at[j]], kbuf.at[slot], gsem[slot])
        pltpu.async_copy(qv_hbm.at[sidx.at[j]], qvbuf.at[slot], gsem[slot])

    def _wait_gather(slot):
        pltpu.make_async_copy(k_hbm.at[didx.at[0]], kbuf.at[slot],
                              gsem[slot]).wait()
        pltpu.make_async_copy(qv_hbm.at[sidx.at[0]], qvbuf.at[slot],
                              gsem[slot]).wait()

    def _compute(gslot, mslot):
        hh = h // 2

        def _unpack(w):
            lo = lax.bitcast_convert_type(w << 16, jnp.float32)
            hi = lax.bitcast_convert_type(w & jnp.int32(-65536), jnp.float32)
            return lo, hi

        def _edge(t, _):
            for w in range(hh // _L):
                klo = kbuf[gslot, t, pl.ds(w * _L, _L)]
                khi = kbuf[gslot, t, pl.ds(hh + w * _L, _L)]
                qlo, qhi = _unpack(qvbuf[gslot, t, pl.ds(w * _L, _L)])
                vlo, vhi = _unpack(qvbuf[gslot, t, pl.ds(hh + w * _L, _L)])
                msgbuf[mslot, t, pl.ds(w * _L, _L)] = (
                    jnp.maximum(klo + qlo, 0.0) * vlo)
                msgbuf[mslot, t, pl.ds(hh + w * _L, _L)] = (
                    jnp.maximum(khi + qhi, 0.0) * vhi)
            return 0
        lax.fori_loop(0, _CH, _edge, 0)

    def _issue_scatter(j, slot):
        pltpu.async_copy(msgbuf.at[slot], agg_sh.at[didx.at[j]], ssem[slot],
                         add=True)

    def _wait_scatter(slot):
        pltpu.make_async_copy(msgbuf.at[slot], agg_sh.at[didx.at[0]],
                              ssem[slot]).wait()

    def _super(si, _):
        ca = pltpu.async_copy(src_hbm.at[wid, si], sidx, i0)
        cb = pltpu.async_copy(dst_hbm.at[wid, si], didx, i0)
        ca.wait()
        cb.wait()
        _issue_gather(0, 0)
        _issue_gather(1, 1)
        for c in range(_NCHK):
            if c + 2 < _NCHK:
                _issue_gather(c + 2, (c + 2) % _GD)
            _wait_gather(c % _GD)
            m = c % 2
            if c >= 2:
                _wait_scatter(m)
            _compute(c % _GD, m)
            _issue_scatter(c, m)
        _wait_scatter(0)
        _wait_scatter(1)
        return 0
    lax.fori_loop(0, nsc, _super, 0)
    plsc.subcore_barrier()

    for m in range(rows_pt // _CH):
        pltpu.sync_copy(agg_sh.at[pl.ds(r0 + m * _CH, _CH)], msgbuf.at[0])
        pltpu.sync_copy(msgbuf.at[0], agg_hbm.at[c, pl.ds(r0 + m * _CH, _CH)])


def _sc_edge(src, dst, k_table, qv_table):
    n, h = k_table.shape
    e = src.shape[0]
    n_pad = -(-n // (_NS * _CH)) * (_NS * _CH)
    epw = e // _NW
    nsc = epw // (_NCHK * _CH)
    src4 = src.reshape(_NW, nsc, _NCHK, _CH)
    dst4 = dst.reshape(_NW, nsc, _NCHK, _CH)
    body = functools.partial(_sc_edge_body, n, e, h)
    mesh = plsc.VectorSubcoreMesh(core_axis_name="c", subcore_axis_name="s")
    f = pl.kernel(
        body,
        out_type=jax.ShapeDtypeStruct((_NC, n_pad, h), jnp.float32),
        mesh=mesh,
        scratch_types=[
            pltpu.VMEM((_NCHK, _CH), jnp.int32),
            pltpu.VMEM((_NCHK, _CH), jnp.int32),
            pltpu.VMEM((_GD, _CH, h), jnp.float32),
            pltpu.VMEM((_GD, _CH, h), jnp.int32),
            pltpu.VMEM((2, _CH, h), jnp.float32),
            pltpu.VMEM_SHARED((n_pad, h), jnp.float32),
            pltpu.SemaphoreType.DMA,
            pltpu.SemaphoreType.DMA,
            pltpu.SemaphoreType.DMA,
            pltpu.SemaphoreType.DMA,
            pltpu.SemaphoreType.DMA,
            pltpu.SemaphoreType.DMA,
        ],
    )
    return f(src4, dst4, k_table, qv_table)


def kernel(x, edge_index, W_key, b_key, W_query, b_query, W_value, b_value,
           W_skip, b_skip, bias, W_lin, b_lin):
    w_qv = jnp.concatenate([W_query, W_value], axis=1)
    b_qv = jnp.concatenate([b_query, b_value], axis=0)
    k_table, qv_table = _pre(x, W_key, w_qv, b_key, b_qv, blk=2000)
    agg = _sc_edge(edge_index[0], edge_index[1], k_table, qv_table)
    return _post(x, agg, W_skip, W_lin, b_skip + bias, b_lin, blk=2000)


# speedup vs baseline: 1.5098x; 1.0088x over previous
import functools

import jax
import jax.numpy as jnp
from jax import lax
from jax.experimental import pallas as pl
from jax.experimental.pallas import tpu as pltpu
from jax.experimental.pallas import tpu_sc as plsc

_NC = 2
_NS = 16
_NW = _NC * _NS
_L = 16
_CH = 40


def _pack_pair(a, b):
    au = lax.bitcast_convert_type(a.astype(jnp.bfloat16), jnp.uint16)
    bu = lax.bitcast_convert_type(b.astype(jnp.bfloat16), jnp.uint16)
    w = au.astype(jnp.uint32) | (bu.astype(jnp.uint32) << 16)
    return lax.bitcast_convert_type(w, jnp.int32)


def _pre_body(x_ref, wk_ref, wqv_ref, bk_ref, bqv_ref, k_ref, qv_ref):
    xb = x_ref[...]
    h = wk_ref.shape[1]
    k_ref[...] = (
        jnp.dot(xb, wk_ref[...], preferred_element_type=jnp.float32)
        + bk_ref[...])
    qv = (jnp.dot(xb, wqv_ref[...], preferred_element_type=jnp.float32)
          + bqv_ref[...])
    qv_ref[...] = jnp.concatenate(
        [_pack_pair(qv[:, :h // 2], qv[:, h // 2:h]),
         _pack_pair(qv[:, h:h + h // 2], qv[:, h + h // 2:])], axis=1)


def _pre(x, w_key, w_qv, b_key, b_qv, blk):
    n, d = x.shape
    h = w_key.shape[1]
    grid = (n // blk,)
    return pl.pallas_call(
        _pre_body,
        grid=grid,
        in_specs=[
            pl.BlockSpec((blk, d), lambda i: (i, 0)),
            pl.BlockSpec((d, h), lambda i: (0, 0)),
            pl.BlockSpec((d, 2 * h), lambda i: (0, 0)),
            pl.BlockSpec((1, h), lambda i: (0, 0)),
            pl.BlockSpec((1, 2 * h), lambda i: (0, 0)),
        ],
        out_specs=[
            pl.BlockSpec((blk, h), lambda i: (i, 0)),
            pl.BlockSpec((blk, h), lambda i: (i, 0)),
        ],
        out_shape=[
            jax.ShapeDtypeStruct((n, h), jnp.float32),
            jax.ShapeDtypeStruct((n, h), jnp.int32),
        ],
    )(x, w_key, w_qv, b_key[None, :], b_qv[None, :])


def _post_body(x_ref, a0_ref, a1_ref, ws_ref, wl_ref, bs_ref, bl_ref, o_ref):
    a = (
        a0_ref[0]
        + a1_ref[0]
        + jnp.dot(x_ref[...], ws_ref[...], preferred_element_type=jnp.float32)
        + bs_ref[...]
    )
    a = jnp.where(a > 0, a, 0.01 * a)
    o_ref[...] = (
        jnp.dot(a, wl_ref[...], preferred_element_type=jnp.float32) + bl_ref[...]
    )


def _post(x, agg, w_skip, w_lin, b_s, b_l, blk):
    n, d = x.shape
    h = w_skip.shape[1]
    grid = (n // blk,)
    return pl.pallas_call(
        _post_body,
        grid=grid,
        in_specs=[
            pl.BlockSpec((blk, d), lambda i: (i, 0)),
            pl.BlockSpec((1, blk, h), lambda i: (0, i, 0)),
            pl.BlockSpec((1, blk, h), lambda i: (1, i, 0)),
            pl.BlockSpec((d, h), lambda i: (0, 0)),
            pl.BlockSpec((h, h), lambda i: (0, 0)),
            pl.BlockSpec((1, h), lambda i: (0, 0)),
            pl.BlockSpec((1, h), lambda i: (0, 0)),
        ],
        out_specs=pl.BlockSpec((blk, h), lambda i: (i, 0)),
        out_shape=jax.ShapeDtypeStruct((n, h), jnp.float32),
    )(x, agg, agg, w_skip, w_lin, b_s[None, :], b_l[None, :])


_NCHK = 25
_GD = 3


def _sc_edge_body(n, e, h, src_hbm, dst_hbm, k_hbm, qv_hbm, agg_hbm,
                  sidx, didx, kbuf, qvbuf, msgbuf, agg_sh,
                  g0, g1, g2, s0, s1, i0):
    c = lax.axis_index("c")
    s = lax.axis_index("s")
    wid = c * _NS + s
    epw = e // _NW
    nsc = epw // (_NCHK * _CH)
    n_pad = agg_hbm.shape[1]
    rows_pt = n_pad // _NS
    r0 = s * rows_pt
    h8 = h // _L
    gsem = (g0, g1, g2)
    ssem = (s0, s1)

    def _zrow(i, _):
        for w in range(h8):
            msgbuf[0, i, pl.ds(w * _L, _L)] = jnp.zeros((_L,), jnp.float32)
        return 0
    lax.fori_loop(0, _CH, _zrow, 0)
    for m in range(rows_pt // _CH):
        pltpu.sync_copy(msgbuf.at[0], agg_sh.at[pl.ds(r0 + m * _CH, _CH)])
    plsc.subcore_barrier()

    def _issue_gather(j, slot):
        pltpu.async_copy(k_hbm.at[didx.at[j]], kbuf.at[slot], gsem[slot])
        pltpu.async_copy(qv_hbm.at[sidx.at[j]], qvbuf.at[slot], gsem[slot])

    def _wait_gather(slot):
        pltpu.make_async_copy(k_hbm.at[didx.at[0]], kbuf.at[slot],
                              gsem[slot]).wait()
        pltpu.make_async_copy(qv_hbm.at[sidx.at[0]], qvbuf.at[slot],
                              gsem[slot]).wait()

    def _compute(gslot, mslot):
        hh = h // 2

        def _unpack(w):
            lo = lax.bitcast_convert_type(w << 16, jnp.float32)
            hi = lax.bitcast_convert_type(w & jnp.int32(-65536), jnp.float32)
            return lo, hi

        def _edge(t, _):
            for w in range(hh // _L):
                klo = kbuf[gslot, t, pl.ds(w * _L, _L)]
                khi = kbuf[gslot, t, pl.ds(hh + w * _L, _L)]
                qlo, qhi = _unpack(qvbuf[gslot, t, pl.ds(w * _L, _L)])
                vlo, vhi = _unpack(qvbuf[gslot, t, pl.ds(hh + w * _L, _L)])
                msgbuf[mslot, t, pl.ds(w * _L, _L)] = (
                    jnp.maximum(klo + qlo, 0.0) * vlo)
                msgbuf[mslot, t, pl.ds(hh + w * _L, _L)] = (
                    jnp.maximum(khi + qhi, 0.0) * vhi)
            return 0
        lax.fori_loop(0, _CH, _edge, 0)

    def _issue_scatter(j, slot):
        pltpu.async_copy(msgbuf.at[slot], agg_sh.at[didx.at[j]], ssem[slot],
                         add=True)

    def _wait_scatter(slot):
        pltpu.make_async_copy(msgbuf.at[slot], agg_sh.at[didx.at[0]],
                              ssem[slot]).wait()

    def _super(si, _):
        ca = pltpu.async_copy(src_hbm.at[wid, si], sidx, i0)
        cb = pltpu.async_copy(dst_hbm.at[wid, si], didx, i0)
        ca.wait()
        cb.wait()
        _issue_gather(0, 0)
        _issue_gather(1, 1)
        for c in range(_NCHK):
            if c + 2 < _NCHK:
                _issue_gather(c + 2, (c + 2) % _GD)
            _wait_gather(c % _GD)
            m = c % 2
            if c >= 2:
                _wait_scatter(m)
            _compute(c % _GD, m)
            _issue_scatter(c, m)
        _wait_scatter(0)
        _wait_scatter(1)
        return 0
    lax.fori_loop(0, nsc, _super, 0)
    plsc.subcore_barrier()

    pltpu.sync_copy(agg_sh.at[pl.ds(r0, rows_pt)],
                    agg_hbm.at[c, pl.ds(r0, rows_pt)])


def _sc_edge(src, dst, k_table, qv_table):
    n, h = k_table.shape
    e = src.shape[0]
    n_pad = -(-n // (_NS * _CH)) * (_NS * _CH)
    epw = e // _NW
    nsc = epw // (_NCHK * _CH)
    src4 = src.reshape(_NW, nsc, _NCHK, _CH)
    dst4 = dst.reshape(_NW, nsc, _NCHK, _CH)
    body = functools.partial(_sc_edge_body, n, e, h)
    mesh = plsc.VectorSubcoreMesh(core_axis_name="c", subcore_axis_name="s")
    f = pl.kernel(
        body,
        out_type=jax.ShapeDtypeStruct((_NC, n_pad, h), jnp.float32),
        mesh=mesh,
        scratch_types=[
            pltpu.VMEM((_NCHK, _CH), jnp.int32),
            pltpu.VMEM((_NCHK, _CH), jnp.int32),
            pltpu.VMEM((_GD, _CH, h), jnp.float32),
            pltpu.VMEM((_GD, _CH, h), jnp.int32),
            pltpu.VMEM((2, _CH, h), jnp.float32),
            pltpu.VMEM_SHARED((n_pad, h), jnp.float32),
            pltpu.SemaphoreType.DMA,
            pltpu.SemaphoreType.DMA,
            pltpu.SemaphoreType.DMA,
            pltpu.SemaphoreType.DMA,
            pltpu.SemaphoreType.DMA,
            pltpu.SemaphoreType.DMA,
        ],
    )
    return f(src4, dst4, k_table, qv_table)


def kernel(x, edge_index, W_key, b_key, W_query, b_query, W_value, b_value,
           W_skip, b_skip, bias, W_lin, b_lin):
    w_qv = jnp.concatenate([W_query, W_value], axis=1)
    b_qv = jnp.concatenate([b_query, b_value], axis=0)
    k_table, qv_table = _pre(x, W_key, w_qv, b_key, b_qv, blk=2000)
    agg = _sc_edge(edge_index[0], edge_index[1], k_table, qv_table)
    return _post(x, agg, W_skip, W_lin, b_skip + bias, b_lin, blk=2000)
